# Initial kernel scaffold; baseline (speedup 1.0000x reference)
import jax
import jax.numpy as jnp
from jax.experimental import pallas as pl


def kernel(sequence, embedding_matrix):
    raise NotImplementedError("write your pallas kernel here")


# speedup vs baseline: 3.2286x; 3.2286x over previous
import functools

import jax
import jax.numpy as jnp
from jax import lax
from jax.experimental import pallas as pl
from jax.experimental.pallas import tpu as pltpu
from jax.experimental.pallas import tpu_sc as plsc

_CHUNK = 128
_DPAD = 128


@functools.cache
def _make_gather(B: int, D: int):
    info = plsc.get_sparse_core_info()
    nw = info.num_cores * info.num_subcores
    b_per_w = B // nw
    n_chunks = b_per_w // _CHUNK
    mesh = plsc.VectorSubcoreMesh(core_axis_name="c", subcore_axis_name="s")

    @functools.partial(
        pl.kernel,
        out_type=jax.ShapeDtypeStruct((B, _DPAD), jnp.float32),
        mesh=mesh,
        scratch_types=[
            pltpu.VMEM((_CHUNK,), jnp.int32),
            pltpu.VMEM((_CHUNK, _DPAD), jnp.float32),
            pltpu.SemaphoreType.DMA,
        ],
    )
    def gather_kernel(table_hbm, idx_hbm, out_hbm, idx_v, rows_v, sem):
        wid = lax.axis_index("s") * info.num_cores + lax.axis_index("c")
        base = wid * b_per_w

        def body(c, carry):
            off = base + c * _CHUNK
            pltpu.sync_copy(idx_hbm.at[pl.ds(off, _CHUNK)], idx_v)
            pltpu.async_copy(table_hbm.at[idx_v], rows_v, sem).wait()
            pltpu.sync_copy(rows_v, out_hbm.at[pl.ds(off, _CHUNK)])
            return carry

        lax.fori_loop(0, n_chunks, body, 0)

    return gather_kernel


def kernel(sequence, embedding_matrix):
    seq_shape = sequence.shape
    B = seq_shape[0] * seq_shape[1]
    D = embedding_matrix.shape[1]
    idx = sequence.reshape(B).astype(jnp.int32)
    table_p = jnp.pad(embedding_matrix, ((0, 0), (0, _DPAD - D)))
    out = _make_gather(B, D)(table_p, idx)
    return out[:, :D].reshape(seq_shape + (D,))


# speedup vs baseline: 4.2471x; 1.3155x over previous
import functools

import jax
import jax.numpy as jnp
from jax import lax
from jax.experimental import pallas as pl
from jax.experimental.pallas import tpu as pltpu
from jax.experimental.pallas import tpu_sc as plsc

_CHUNK = 128
_DPAD = 128


@functools.cache
def _make_gather(B: int, D: int):
    info = plsc.get_sparse_core_info()
    nw = info.num_cores * info.num_subcores
    b_per_w = B // nw
    n_chunks = b_per_w // _CHUNK
    mesh = plsc.VectorSubcoreMesh(core_axis_name="c", subcore_axis_name="s")

    nbuf = 4
    n_groups = n_chunks // nbuf

    @functools.partial(
        pl.kernel,
        out_type=jax.ShapeDtypeStruct((B, _DPAD), jnp.float32),
        mesh=mesh,
        scratch_types=[
            pltpu.VMEM((n_chunks, _CHUNK), jnp.int32),
            pltpu.VMEM((nbuf, _CHUNK, _DPAD), jnp.float32),
            [pltpu.SemaphoreType.DMA] * nbuf,
            [pltpu.SemaphoreType.DMA] * nbuf,
        ],
    )
    def gather_kernel(table_hbm, idx_hbm, out_hbm, idx_v, rows_v, gsems, wsems):
        wid = lax.axis_index("s") * info.num_cores + lax.axis_index("c")
        base = wid * b_per_w

        pltpu.sync_copy(idx_hbm.at[pl.ds(wid * n_chunks, n_chunks)], idx_v)

        def wait_write(b):
            pltpu.make_async_copy(
                rows_v.at[b], out_hbm.at[pl.ds(base, _CHUNK)], wsems[b]
            ).wait()

        def body(g, carry):
            c0 = g * nbuf
            for b in range(nbuf):
                @pl.when(g > 0)
                def _():
                    wait_write(b)
                pltpu.async_copy(
                    table_hbm.at[idx_v.at[c0 + b]], rows_v.at[b], gsems[b]
                )
            for b in range(nbuf):
                pltpu.make_async_copy(
                    table_hbm.at[idx_v.at[c0 + b]], rows_v.at[b], gsems[b]
                ).wait()
                off = base + (c0 + b) * _CHUNK
                pltpu.async_copy(
                    rows_v.at[b], out_hbm.at[pl.ds(off, _CHUNK)], wsems[b]
                )
            return carry

        lax.fori_loop(0, n_groups, body, 0)
        for b in range(nbuf):
            wait_write(b)

    return gather_kernel


def kernel(sequence, embedding_matrix):
    seq_shape = sequence.shape
    B = seq_shape[0] * seq_shape[1]
    D = embedding_matrix.shape[1]
    idx = sequence.reshape(B // _CHUNK, _CHUNK).astype(jnp.int32)
    table_p = jnp.pad(embedding_matrix, ((0, 0), (0, _DPAD - D)))
    out = _make_gather(B, D)(table_p, idx)
    return out[:, :D].reshape(seq_shape + (D,))


# speedup vs baseline: 4.8070x; 1.1318x over previous
import functools

import jax
import jax.numpy as jnp
from jax import lax
from jax.experimental import pallas as pl
from jax.experimental.pallas import tpu as pltpu
from jax.experimental.pallas import tpu_sc as plsc

_CHUNK = 64
_DPAD = 128
_NBUF = 4


@functools.cache
def _make_pad(V: int, D: int):
    rows = 2048
    grid = (V + rows - 1) // rows

    def pad_block(x_ref, o_ref):
        o_ref[:, :D] = x_ref[...]
        o_ref[:, D:] = jnp.zeros_like(o_ref[:, D:])

    return pl.pallas_call(
        pad_block,
        grid=(grid,),
        in_specs=[pl.BlockSpec((rows, D), lambda i: (i, 0))],
        out_specs=pl.BlockSpec((rows, _DPAD), lambda i: (i, 0)),
        out_shape=jax.ShapeDtypeStruct((V, _DPAD), jnp.float32),
    )


@functools.cache
def _make_gather(B: int, D: int):
    info = plsc.get_sparse_core_info()
    nw = info.num_cores * info.num_subcores
    b_per_w = B // nw
    n_chunks = b_per_w // _CHUNK
    n_groups = n_chunks // _NBUF
    mesh = plsc.VectorSubcoreMesh(core_axis_name="c", subcore_axis_name="s")

    @functools.partial(
        pl.kernel,
        out_type=jax.ShapeDtypeStruct((B, D), jnp.float32),
        mesh=mesh,
        scratch_types=[
            pltpu.VMEM((b_per_w,), jnp.int32),
            [pltpu.VMEM((_CHUNK, _DPAD), jnp.float32)] * _NBUF,
            [pltpu.VMEM((_CHUNK, D), jnp.float32)] * _NBUF,
            [pltpu.SemaphoreType.DMA] * _NBUF,
            [pltpu.SemaphoreType.DMA] * _NBUF,
        ],
    )
    def gather_kernel(table_hbm, idx_hbm, out_hbm, idx_v, wide, packed,
                      gsems, wsems):
        wid = lax.axis_index("s") * info.num_cores + lax.axis_index("c")
        base = wid * b_per_w

        pltpu.sync_copy(idx_hbm.at[pl.ds(base, b_per_w)], idx_v)

        def wait_write(b):
            pltpu.make_async_copy(
                packed[b], out_hbm.at[pl.ds(base, _CHUNK)], wsems[b]
            ).wait()

        def compact(b):
            def rows4(r4, carry):
                r = r4 * 4
                for dr in range(4):
                    for k in (0, 16, 32, 48, 64, 80, D - 16):
                        packed[b][r + dr, pl.ds(k, 16)] = (
                            wide[b][r + dr, pl.ds(k, 16)]
                        )
                return carry

            lax.fori_loop(0, _CHUNK // 4, rows4, 0)

        def body(g, carry):
            c0 = g * _NBUF
            for b in range(_NBUF):
                @pl.when(g > 0)
                def _():
                    wait_write(b)
                pltpu.async_copy(
                    table_hbm.at[idx_v.at[pl.ds((c0 + b) * _CHUNK, _CHUNK)]],
                    wide[b],
                    gsems[b],
                )
            for b in range(_NBUF):
                pltpu.make_async_copy(
                    table_hbm.at[idx_v.at[pl.ds((c0 + b) * _CHUNK, _CHUNK)]],
                    wide[b],
                    gsems[b],
                ).wait()
                compact(b)
                off = base + (c0 + b) * _CHUNK
                pltpu.async_copy(
                    packed[b], out_hbm.at[pl.ds(off, _CHUNK)], wsems[b]
                )
            return carry

        lax.fori_loop(0, n_groups, body, 0)
        for b in range(_NBUF):
            wait_write(b)

    return gather_kernel


def kernel(sequence, embedding_matrix):
    seq_shape = sequence.shape
    B = seq_shape[0] * seq_shape[1]
    V, D = embedding_matrix.shape
    idx = sequence.reshape(B).astype(jnp.int32)
    table_p = _make_pad(V, D)(embedding_matrix)
    out = _make_gather(B, D)(table_p, idx)
    return out.reshape(seq_shape + (D,))
